# baseline (speedup 1.0000x reference)
import functools

import jax
import jax.numpy as jnp
from jax import lax
from jax.experimental import pallas as pl
from jax.experimental.pallas import tpu as pltpu
from jax.experimental.pallas import tpu_sc as plsc

_NUM_USERS = 6000
_NUM_ITEMS = 4000
_N_TOTAL = _NUM_USERS + _NUM_ITEMS
_D = 64
_BM = 200


def _mm_body(a_ref, x_ref, p_ref, o_ref, acc_ref):
    o = jnp.dot(a_ref[...], x_ref[...], preferred_element_type=jnp.float32)
    o_ref[...] = o
    acc_ref[...] = p_ref[...] + o


def _mm(adj, x, prev):
    return pl.pallas_call(
        _mm_body,
        grid=(_N_TOTAL // _BM,),
        in_specs=[
            pl.BlockSpec((_BM, _N_TOTAL), lambda i: (i, 0)),
            pl.BlockSpec((_N_TOTAL, _D), lambda i: (0, 0)),
            pl.BlockSpec((_BM, _D), lambda i: (i, 0)),
        ],
        out_specs=[
            pl.BlockSpec((_BM, _D), lambda i: (i, 0)),
            pl.BlockSpec((_BM, _D), lambda i: (i, 0)),
        ],
        out_shape=[
            jax.ShapeDtypeStruct((_N_TOTAL, _D), jnp.float32),
            jax.ShapeDtypeStruct((_N_TOTAL, _D), jnp.float32),
        ],
    )(adj, x, prev)


def _sc_gather(table, idx):
    b = idx.shape[0]
    info = plsc.get_sparse_core_info()
    nw = info.num_cores * info.num_subcores
    b_per_w = b // nw
    mesh = plsc.VectorSubcoreMesh(core_axis_name="c", subcore_axis_name="s")

    @functools.partial(
        pl.kernel,
        mesh=mesh,
        compiler_params=pltpu.CompilerParams(use_tc_tiling_on_sc=False),
        out_type=jax.ShapeDtypeStruct((b, _D), jnp.float32),
        scratch_types=[
            pltpu.VMEM((b_per_w,), jnp.int32),
            pltpu.VMEM((b_per_w, _D), jnp.float32),
            pltpu.SemaphoreType.DMA,
        ],
    )
    def k(table_hbm, idx_hbm, out_hbm, idx_v, rows_v, sem):
        wid = lax.axis_index("s") * info.num_cores + lax.axis_index("c")
        base = wid * b_per_w
        pltpu.sync_copy(idx_hbm.at[pl.ds(base, b_per_w)], idx_v)
        pltpu.async_copy(table_hbm.at[idx_v], rows_v, sem).wait()
        pltpu.sync_copy(rows_v, out_hbm.at[pl.ds(base, b_per_w)])

    return k(table, idx)


_BP = 256


def _dot_body(u_ref, i_ref, o_ref):
    o_ref[...] = jnp.sum(u_ref[...] * i_ref[...], axis=1) * (1.0 / 16.0)


def _dot(gu, gi):
    n = gu.shape[0]
    return pl.pallas_call(
        _dot_body,
        grid=(n // _BP,),
        in_specs=[
            pl.BlockSpec((_BP, _D), lambda i: (i, 0)),
            pl.BlockSpec((_BP, _D), lambda i: (i, 0)),
        ],
        out_specs=pl.BlockSpec((_BP,), lambda i: (i,)),
        out_shape=jax.ShapeDtypeStruct((n,), jnp.float32),
    )(gu, gi)


def kernel(adj, users, items, user_emb, item_emb):
    e0 = jnp.concatenate([user_emb, item_emb], axis=0)
    x1, a1 = _mm(adj, e0, e0)
    x2, a2 = _mm(adj, x1, a1)
    _, s = _mm(adj, x2, a2)
    idx = jnp.concatenate(
        [users.astype(jnp.int32), items.astype(jnp.int32) + _NUM_USERS]
    )
    rows = _sc_gather(s, idx)
    gamma = _dot(rows[: users.shape[0]], rows[users.shape[0]:])
    return gamma


# speedup vs baseline: 1.1291x; 1.1291x over previous
import functools

import jax
import jax.numpy as jnp
from jax import lax
from jax.experimental import pallas as pl
from jax.experimental.pallas import tpu as pltpu
from jax.experimental.pallas import tpu_sc as plsc

_NUM_USERS = 6000
_NUM_ITEMS = 4000
_N_TOTAL = _NUM_USERS + _NUM_ITEMS
_D = 64
_BM = 200


def _mm_body(a_ref, x_ref, p_ref, o_ref, acc_ref):
    o = jnp.dot(a_ref[...], x_ref[...], preferred_element_type=jnp.float32)
    o_ref[...] = o
    acc_ref[...] = p_ref[...] + o


def _mm(adj, x, prev):
    return pl.pallas_call(
        _mm_body,
        grid=(_N_TOTAL // _BM,),
        in_specs=[
            pl.BlockSpec((_BM, _N_TOTAL), lambda i: (i, 0)),
            pl.BlockSpec((_N_TOTAL, _D), lambda i: (0, 0)),
            pl.BlockSpec((_BM, _D), lambda i: (i, 0)),
        ],
        out_specs=[
            pl.BlockSpec((_BM, _D), lambda i: (i, 0)),
            pl.BlockSpec((_BM, _D), lambda i: (i, 0)),
        ],
        out_shape=[
            jax.ShapeDtypeStruct((_N_TOTAL, _D), jnp.float32),
            jax.ShapeDtypeStruct((_N_TOTAL, _D), jnp.float32),
        ],
    )(adj, x, prev)


def _sc_gather(table, idx):
    b = idx.shape[0]
    info = plsc.get_sparse_core_info()
    nw = info.num_cores * info.num_subcores
    b_per_w = b // nw
    mesh = plsc.VectorSubcoreMesh(core_axis_name="c", subcore_axis_name="s")

    @functools.partial(
        pl.kernel,
        mesh=mesh,
        compiler_params=pltpu.CompilerParams(use_tc_tiling_on_sc=False),
        out_type=jax.ShapeDtypeStruct((b, _D), jnp.float32),
        scratch_types=[
            pltpu.VMEM((b_per_w,), jnp.int32),
            pltpu.VMEM((b_per_w, _D), jnp.float32),
            pltpu.SemaphoreType.DMA,
        ],
    )
    def k(table_hbm, idx_hbm, out_hbm, idx_v, rows_v, sem):
        wid = lax.axis_index("s") * info.num_cores + lax.axis_index("c")
        base = wid * b_per_w
        pltpu.sync_copy(idx_hbm.at[pl.ds(base, b_per_w)], idx_v)
        pltpu.async_copy(table_hbm.at[idx_v], rows_v, sem).wait()
        pltpu.sync_copy(rows_v, out_hbm.at[pl.ds(base, b_per_w)])

    return k(table, idx)


_BR = 64


def _rows_mm_body(idx_ref, x2_ref, adj_ref, o_ref, buf0, buf1, sem0, sem1):
    i = pl.program_id(0)
    nsteps = pl.num_programs(0)

    def issue(step, buf, sem):
        def body(j, carry):
            row = idx_ref[step * _BR + j]
            pltpu.make_async_copy(
                adj_ref.at[pl.ds(row, 1), :],
                buf.at[pl.ds(j, 1), :],
                sem,
            ).start()
            return carry

        lax.fori_loop(0, _BR, body, 0)

    def wait(buf, sem):
        pltpu.make_async_copy(adj_ref.at[pl.ds(0, _BR), :], buf, sem).wait()

    @pl.when(i == 0)
    def _():
        issue(0, buf0, sem0)

    @pl.when(jnp.logical_and(i + 1 < nsteps, (i + 1) % 2 == 0))
    def _():
        issue(i + 1, buf0, sem0)

    @pl.when(jnp.logical_and(i + 1 < nsteps, (i + 1) % 2 == 1))
    def _():
        issue(i + 1, buf1, sem1)

    @pl.when(i % 2 == 0)
    def _():
        wait(buf0, sem0)
        o_ref[...] = jnp.dot(
            buf0[...], x2_ref[...], preferred_element_type=jnp.float32
        )

    @pl.when(i % 2 == 1)
    def _():
        wait(buf1, sem1)
        o_ref[...] = jnp.dot(
            buf1[...], x2_ref[...], preferred_element_type=jnp.float32
        )


def _rows_mm(adj, idx, x2):
    n = idx.shape[0]
    return pl.pallas_call(
        _rows_mm_body,
        grid=(n // _BR,),
        in_specs=[
            pl.BlockSpec(memory_space=pltpu.SMEM),
            pl.BlockSpec((_N_TOTAL, _D), lambda i: (0, 0)),
            pl.BlockSpec(memory_space=pl.ANY),
        ],
        out_specs=pl.BlockSpec((_BR, _D), lambda i: (i, 0)),
        out_shape=jax.ShapeDtypeStruct((n, _D), jnp.float32),
        scratch_shapes=[
            pltpu.VMEM((_BR, _N_TOTAL), jnp.float32),
            pltpu.VMEM((_BR, _N_TOTAL), jnp.float32),
            pltpu.SemaphoreType.DMA,
            pltpu.SemaphoreType.DMA,
        ],
    )(idx, x2, adj)


_BP = 256


def _dot_body(gu_ref, gi_ref, xu_ref, xi_ref, o_ref):
    su = gu_ref[...] + xu_ref[...]
    si = gi_ref[...] + xi_ref[...]
    o_ref[...] = jnp.sum(su * si, axis=1) * (1.0 / 16.0)


def _dot(g, x3, npairs):
    off = npairs // _BP
    return pl.pallas_call(
        _dot_body,
        grid=(npairs // _BP,),
        in_specs=[
            pl.BlockSpec((_BP, _D), lambda i: (i, 0)),
            pl.BlockSpec((_BP, _D), lambda i: (i + off, 0)),
            pl.BlockSpec((_BP, _D), lambda i: (i, 0)),
            pl.BlockSpec((_BP, _D), lambda i: (i + off, 0)),
        ],
        out_specs=pl.BlockSpec((_BP,), lambda i: (i,)),
        out_shape=jax.ShapeDtypeStruct((npairs,), jnp.float32),
    )(g, g, x3, x3)


def kernel(adj, users, items, user_emb, item_emb):
    e0 = jnp.concatenate([user_emb, item_emb], axis=0)
    x1, a1 = _mm(adj, e0, e0)
    x2, a2 = _mm(adj, x1, a1)
    idx = jnp.concatenate(
        [users.astype(jnp.int32), items.astype(jnp.int32) + _NUM_USERS]
    )
    g = _sc_gather(a2, idx)
    x3 = _rows_mm(adj, idx, x2)
    return _dot(g, x3, users.shape[0])
